# baseline (speedup 1.0000x reference)
import functools

import jax
import jax.numpy as jnp
from jax import lax
from jax.experimental import pallas as pl
from jax.experimental.pallas import tpu as pltpu
from jax.experimental.pallas import tpu_sc as plsc

_N_CORES = 2
_N_SUBCORES = 16
_N_WORKERS = _N_CORES * _N_SUBCORES
_LANES = 16
_IDX_W = 128


def kernel(A, D, l, i, concepts, guess_prob):
    n_students, n_concepts = A.shape

    chunk = -(-n_students // _N_WORKERS)
    chunk = -(-chunk // _IDX_W) * _IDX_W
    n_sub = chunk // _IDX_W
    batches = [1, 3, 6, 7, 6, 2]
    assert sum(batches) == n_sub

    rc = lax.dynamic_slice_in_dim(concepts, i, 1, axis=1)
    scal_f = jnp.concatenate([
        jnp.asarray(l, jnp.float32).reshape(1),
        jnp.asarray(guess_prob, jnp.float32).reshape(1),
        rc.reshape(2).astype(jnp.float32),
        jnp.zeros((_LANES - 4,), jnp.float32),
    ])

    A_flat = A.reshape(-1)

    mesh = plsc.VectorSubcoreMesh(
        core_axis_name="c", subcore_axis_name="s",
        num_cores=_N_CORES, num_subcores=_N_SUBCORES)

    @functools.partial(
        pl.kernel,
        out_type=jax.ShapeDtypeStruct((n_students,), jnp.float32),
        mesh=mesh,
        compiler_params=pltpu.CompilerParams(
            use_tc_tiling_on_sc=False, needs_layout_passes=False),
        scratch_types=[
            pltpu.VMEM((_LANES,), jnp.float32),
            pltpu.VMEM((n_concepts,), jnp.float32),
            pltpu.VMEM((chunk,), jnp.int32),
            pltpu.VMEM((chunk,), jnp.float32),
            pltpu.VMEM((chunk,), jnp.float32),
            pltpu.SemaphoreType.DMA,
            pltpu.SemaphoreType.DMA,
            pltpu.SemaphoreType.DMA,
            pltpu.SemaphoreType.DMA,
            pltpu.SemaphoreType.DMA,
            pltpu.SemaphoreType.DMA,
            pltpu.SemaphoreType.DMA,
            pltpu.SemaphoreType.DMA,
            pltpu.SemaphoreType.DMA,
        ],
    )
    def run(A_hbm, D_hbm, sf_hbm, out_hbm, sf_v, drow_v, idx_v, a_v, y_v,
            sem_s, sem_d, sg0, sg1, sg2, sg3, sg4, sg5, sem_w):
        sgs = [sg0, sg1, sg2, sg3, sg4, sg5]
        cid = lax.axis_index("c")
        sid = lax.axis_index("s")
        wid = sid * _N_CORES + cid
        base = jnp.minimum(wid * chunk, n_students - chunk)
        base = pl.multiple_of(base, _LANES)

        lane = lax.iota(jnp.int32, _LANES)
        ninf = jnp.float32(-jnp.inf)

        pltpu.async_copy(sf_hbm, sf_v, sem_s).wait()
        fv = sf_v[...]
        lam = jnp.full((_LANES,), jnp.max(jnp.where(lane == 0, fv, ninf)))
        gp = jnp.full((_LANES,), jnp.max(jnp.where(lane == 1, fv, ninf)))
        r = jnp.max(jnp.where(lane == 2, fv, ninf)).astype(jnp.int32)
        c = jnp.max(jnp.where(lane == 3, fv, ninf)).astype(jnp.int32)

        cp_d = pltpu.make_async_copy(D_hbm.at[r], drow_v, sem_d)
        cp_d.start()

        idx0 = (base + lane) * n_concepts + c
        copies = []
        row = 0
        for b, nr in enumerate(batches):
            n_el = nr * _IDX_W
            off = row * _IDX_W
            for g in range(n_el // _LANES):
                j0 = off + g * _LANES
                idx_v[pl.ds(j0, _LANES)] = idx0 + j0 * n_concepts
            cp = pltpu.make_async_copy(
                A_hbm.at[idx_v.at[pl.ds(off, n_el)]],
                a_v.at[pl.ds(off, n_el)],
                sgs[b])
            cp.start()
            copies.append((b, cp))
            row += nr

        cp_d.wait()
        d = plsc.load_gather(drow_v, [jnp.full((_LANES,), c, jnp.int32)])
        one = jnp.full((_LANES,), jnp.float32(1.0))
        b_vec = lam * d

        row = 0
        for b, nr in enumerate(batches):
            for bb, cp in copies:
                if bb == b:
                    cp.wait()
            off = row * _IDX_W
            for g in range(nr * _IDX_W // _LANES):
                j0 = off + g * _LANES
                a = a_v[pl.ds(j0, _LANES)]
                y = jnp.maximum(gp, one - jnp.exp(b_vec - lam * a))
                y_v[pl.ds(j0, _LANES)] = y
            pltpu.make_async_copy(
                y_v.at[pl.ds(off, nr * _IDX_W)],
                out_hbm.at[pl.ds(base + off, nr * _IDX_W)],
                sem_w,
            ).start()
            row += nr

        row = 0
        for nr in batches:
            pltpu.make_async_copy(
                y_v.at[pl.ds(row * _IDX_W, nr * _IDX_W)],
                out_hbm.at[pl.ds(base + row * _IDX_W, nr * _IDX_W)],
                sem_w,
            ).wait()
            row += nr

    return run(A_flat, D, scal_f)


# speedup vs baseline: 1.0150x; 1.0150x over previous
import functools

import jax
import jax.numpy as jnp
from jax import lax
from jax.experimental import pallas as pl
from jax.experimental.pallas import tpu as pltpu
from jax.experimental.pallas import tpu_sc as plsc

_N_CORES = 2
_N_SUBCORES = 16
_N_WORKERS = _N_CORES * _N_SUBCORES
_LANES = 16
_IDX_W = 128


def kernel(A, D, l, i, concepts, guess_prob):
    n_students, n_concepts = A.shape

    chunk = -(-n_students // _N_WORKERS)
    chunk = -(-chunk // _IDX_W) * _IDX_W
    n_sub = chunk // _IDX_W
    batches = [2, 6, 7, 6, 3, 1]
    assert sum(batches) == n_sub

    rc = lax.dynamic_slice_in_dim(concepts, i, 1, axis=1)
    scal_f = jnp.concatenate([
        jnp.asarray(l, jnp.float32).reshape(1),
        jnp.asarray(guess_prob, jnp.float32).reshape(1),
        rc.reshape(2).astype(jnp.float32),
        jnp.zeros((_LANES - 4,), jnp.float32),
    ])

    A_flat = A.reshape(-1)

    mesh = plsc.VectorSubcoreMesh(
        core_axis_name="c", subcore_axis_name="s",
        num_cores=_N_CORES, num_subcores=_N_SUBCORES)

    @functools.partial(
        pl.kernel,
        out_type=jax.ShapeDtypeStruct((n_students,), jnp.float32),
        mesh=mesh,
        compiler_params=pltpu.CompilerParams(
            use_tc_tiling_on_sc=False, needs_layout_passes=False),
        scratch_types=[
            pltpu.VMEM((_LANES,), jnp.float32),
            pltpu.VMEM((n_concepts,), jnp.float32),
            pltpu.VMEM((chunk,), jnp.int32),
            pltpu.VMEM((chunk,), jnp.float32),
            pltpu.VMEM((chunk,), jnp.float32),
            pltpu.SemaphoreType.DMA,
            pltpu.SemaphoreType.DMA,
            pltpu.SemaphoreType.DMA,
            pltpu.SemaphoreType.DMA,
            pltpu.SemaphoreType.DMA,
            pltpu.SemaphoreType.DMA,
            pltpu.SemaphoreType.DMA,
            pltpu.SemaphoreType.DMA,
            pltpu.SemaphoreType.DMA,
        ],
    )
    def run(A_hbm, D_hbm, sf_hbm, out_hbm, sf_v, drow_v, idx_v, a_v, y_v,
            sem_s, sem_d, sg0, sg1, sg2, sg3, sg4, sg5, sem_w):
        sgs = [sg0, sg1, sg2, sg3, sg4, sg5]
        cid = lax.axis_index("c")
        sid = lax.axis_index("s")
        wid = sid * _N_CORES + cid
        base = jnp.minimum(wid * chunk, n_students - chunk)
        base = pl.multiple_of(base, _LANES)

        lane = lax.iota(jnp.int32, _LANES)
        ninf = jnp.float32(-jnp.inf)

        pltpu.async_copy(sf_hbm, sf_v, sem_s).wait()
        fv = sf_v[...]
        lam = jnp.full((_LANES,), jnp.max(jnp.where(lane == 0, fv, ninf)))
        gp = jnp.full((_LANES,), jnp.max(jnp.where(lane == 1, fv, ninf)))
        r = jnp.max(jnp.where(lane == 2, fv, ninf)).astype(jnp.int32)
        c = jnp.max(jnp.where(lane == 3, fv, ninf)).astype(jnp.int32)

        cp_d = pltpu.make_async_copy(D_hbm.at[r], drow_v, sem_d)
        cp_d.start()

        idx0 = (base + lane) * n_concepts + c
        copies = []
        row = 0
        for b, nr in enumerate(batches):
            n_el = nr * _IDX_W
            off = row * _IDX_W

            @plsc.parallel_loop(off // _LANES, (off + n_el) // _LANES,
                                unroll=8)
            def _build(g):
                j0 = g * _LANES
                idx_v[pl.ds(j0, _LANES)] = idx0 + j0 * n_concepts

            cp = pltpu.make_async_copy(
                A_hbm.at[idx_v.at[pl.ds(off, n_el)]],
                a_v.at[pl.ds(off, n_el)],
                sgs[b])
            cp.start()
            copies.append((b, cp))
            row += nr

        cp_d.wait()
        d = plsc.load_gather(drow_v, [jnp.full((_LANES,), c, jnp.int32)])
        one = jnp.full((_LANES,), jnp.float32(1.0))
        b_vec = lam * d

        row = 0
        for b, nr in enumerate(batches):
            for bb, cp in copies:
                if bb == b:
                    cp.wait()
            off = row * _IDX_W
            n_el = nr * _IDX_W

            @plsc.parallel_loop(off // _LANES, (off + n_el) // _LANES,
                                unroll=8)
            def _compute(g):
                j0 = g * _LANES
                a = a_v[pl.ds(j0, _LANES)]
                y = jnp.maximum(gp, one - jnp.exp(b_vec - lam * a))
                y_v[pl.ds(j0, _LANES)] = y
            pltpu.make_async_copy(
                y_v.at[pl.ds(off, nr * _IDX_W)],
                out_hbm.at[pl.ds(base + off, nr * _IDX_W)],
                sem_w,
            ).start()
            row += nr

        row = 0
        for nr in batches:
            pltpu.make_async_copy(
                y_v.at[pl.ds(row * _IDX_W, nr * _IDX_W)],
                out_hbm.at[pl.ds(base + row * _IDX_W, nr * _IDX_W)],
                sem_w,
            ).wait()
            row += nr

    return run(A_flat, D, scal_f)


# speedup vs baseline: 1.0189x; 1.0039x over previous
import functools

import jax
import jax.numpy as jnp
from jax import lax
from jax.experimental import pallas as pl
from jax.experimental.pallas import tpu as pltpu
from jax.experimental.pallas import tpu_sc as plsc

_N_CORES = 2
_N_SUBCORES = 16
_N_WORKERS = _N_CORES * _N_SUBCORES
_LANES = 16
_IDX_W = 128


def kernel(A, D, l, i, concepts, guess_prob):
    n_students, n_concepts = A.shape

    chunk = -(-n_students // _N_WORKERS)
    chunk = -(-chunk // _IDX_W) * _IDX_W
    n_sub = chunk // _IDX_W
    batches = [2, 6, 7, 6, 3, 1]
    assert sum(batches) == n_sub

    rc = lax.dynamic_slice_in_dim(concepts, i, 1, axis=1)
    scal_f = jnp.concatenate([
        jnp.asarray(l, jnp.float32).reshape(1),
        jnp.asarray(guess_prob, jnp.float32).reshape(1),
        rc.reshape(2).astype(jnp.float32),
        jnp.zeros((_LANES - 4,), jnp.float32),
    ])

    A_flat = A.reshape(-1)

    mesh = plsc.VectorSubcoreMesh(
        core_axis_name="c", subcore_axis_name="s",
        num_cores=_N_CORES, num_subcores=_N_SUBCORES)

    @functools.partial(
        pl.kernel,
        out_type=jax.ShapeDtypeStruct((n_students,), jnp.float32),
        mesh=mesh,
        compiler_params=pltpu.CompilerParams(
            use_tc_tiling_on_sc=False, needs_layout_passes=False),
        scratch_types=[
            pltpu.VMEM((_LANES,), jnp.float32),
            pltpu.VMEM((n_concepts,), jnp.float32),
            pltpu.VMEM((chunk,), jnp.int32),
            pltpu.VMEM((chunk,), jnp.float32),
            pltpu.VMEM((chunk,), jnp.float32),
            pltpu.SemaphoreType.DMA,
            pltpu.SemaphoreType.DMA,
            pltpu.SemaphoreType.DMA,
            pltpu.SemaphoreType.DMA,
            pltpu.SemaphoreType.DMA,
            pltpu.SemaphoreType.DMA,
            pltpu.SemaphoreType.DMA,
            pltpu.SemaphoreType.DMA,
            pltpu.SemaphoreType.DMA,
        ],
    )
    def run(A_hbm, D_hbm, sf_hbm, out_hbm, sf_v, drow_v, idx_v, a_v, y_v,
            sem_s, sem_d, sg0, sg1, sg2, sg3, sg4, sg5, sem_w):
        sgs = [sg0, sg1, sg2, sg3, sg4, sg5]
        cid = lax.axis_index("c")
        sid = lax.axis_index("s")
        wid = sid * _N_CORES + cid
        base = jnp.minimum(wid * chunk, n_students - chunk)
        base = pl.multiple_of(base, _LANES)

        lane = lax.iota(jnp.int32, _LANES)
        ninf = jnp.float32(-jnp.inf)

        pltpu.async_copy(sf_hbm, sf_v, sem_s).wait()
        fv = sf_v[...]
        lam = jnp.full((_LANES,), jnp.max(jnp.where(lane == 0, fv, ninf)))
        gp = jnp.full((_LANES,), jnp.max(jnp.where(lane == 1, fv, ninf)))
        r = jnp.max(jnp.where(lane == 2, fv, ninf)).astype(jnp.int32)
        c = jnp.max(jnp.where(lane == 3, fv, ninf)).astype(jnp.int32)

        cp_d = pltpu.make_async_copy(D_hbm.at[r], drow_v, sem_d)
        cp_d.start()

        idx0 = (base + lane) * n_concepts + c
        copies = []
        row = 0
        for b, nr in enumerate(batches):
            n_el = nr * _IDX_W
            off = row * _IDX_W

            @plsc.parallel_loop(off // _LANES, (off + n_el) // _LANES,
                                unroll=4)
            def _build(g):
                j0 = g * _LANES
                idx_v[pl.ds(j0, _LANES)] = idx0 + j0 * n_concepts

            cp = pltpu.make_async_copy(
                A_hbm.at[idx_v.at[pl.ds(off, n_el)]],
                a_v.at[pl.ds(off, n_el)],
                sgs[b])
            cp.start()
            copies.append((b, cp))
            row += nr

        cp_d.wait()
        d = plsc.load_gather(drow_v, [jnp.full((_LANES,), c, jnp.int32)])
        one = jnp.full((_LANES,), jnp.float32(1.0))
        b_vec = lam * d

        row = 0
        for b, nr in enumerate(batches):
            for bb, cp in copies:
                if bb == b:
                    cp.wait()
            off = row * _IDX_W
            n_el = nr * _IDX_W

            @plsc.parallel_loop(off // _LANES, (off + n_el) // _LANES,
                                unroll=4)
            def _compute(g):
                j0 = g * _LANES
                a = a_v[pl.ds(j0, _LANES)]
                y = jnp.maximum(gp, one - jnp.exp(b_vec - lam * a))
                y_v[pl.ds(j0, _LANES)] = y
            pltpu.make_async_copy(
                y_v.at[pl.ds(off, nr * _IDX_W)],
                out_hbm.at[pl.ds(base + off, nr * _IDX_W)],
                sem_w,
            ).start()
            row += nr

        row = 0
        for nr in batches:
            pltpu.make_async_copy(
                y_v.at[pl.ds(row * _IDX_W, nr * _IDX_W)],
                out_hbm.at[pl.ds(base + row * _IDX_W, nr * _IDX_W)],
                sem_w,
            ).wait()
            row += nr

    return run(A_flat, D, scal_f)


# speedup vs baseline: 1.0214x; 1.0024x over previous
import functools

import jax
import jax.numpy as jnp
from jax import lax
from jax.experimental import pallas as pl
from jax.experimental.pallas import tpu as pltpu
from jax.experimental.pallas import tpu_sc as plsc

_N_CORES = 2
_N_SUBCORES = 16
_N_WORKERS = _N_CORES * _N_SUBCORES
_LANES = 16
_IDX_W = 128


def kernel(A, D, l, i, concepts, guess_prob):
    n_students, n_concepts = A.shape

    chunk = -(-n_students // _N_WORKERS)
    chunk = -(-chunk // _IDX_W) * _IDX_W
    n_sub = chunk // _IDX_W
    batches = [2, 6, 7, 6, 3, 1]
    assert sum(batches) == n_sub

    rc = lax.dynamic_slice_in_dim(concepts, i, 1, axis=1)
    scal_f = jnp.concatenate([
        jnp.asarray(l, jnp.float32).reshape(1),
        jnp.asarray(guess_prob, jnp.float32).reshape(1),
        rc.reshape(2).astype(jnp.float32),
        jnp.zeros((_LANES - 4,), jnp.float32),
    ])

    A_flat = A.reshape(-1)

    mesh = plsc.VectorSubcoreMesh(
        core_axis_name="c", subcore_axis_name="s",
        num_cores=_N_CORES, num_subcores=_N_SUBCORES)

    @functools.partial(
        pl.kernel,
        out_type=jax.ShapeDtypeStruct((n_students,), jnp.float32),
        mesh=mesh,
        compiler_params=pltpu.CompilerParams(
            use_tc_tiling_on_sc=False, needs_layout_passes=False),
        scratch_types=[
            pltpu.VMEM((_LANES,), jnp.float32),
            pltpu.VMEM((n_concepts,), jnp.float32),
            pltpu.VMEM((chunk,), jnp.int32),
            pltpu.VMEM((chunk,), jnp.float32),
            pltpu.VMEM((chunk,), jnp.float32),
            pltpu.SemaphoreType.DMA,
            pltpu.SemaphoreType.DMA,
            pltpu.SemaphoreType.DMA,
            pltpu.SemaphoreType.DMA,
            pltpu.SemaphoreType.DMA,
            pltpu.SemaphoreType.DMA,
            pltpu.SemaphoreType.DMA,
            pltpu.SemaphoreType.DMA,
            pltpu.SemaphoreType.DMA,
        ],
    )
    def run(A_hbm, D_hbm, sf_hbm, out_hbm, sf_v, drow_v, idx_v, a_v, y_v,
            sem_s, sem_d, sg0, sg1, sg2, sg3, sg4, sg5, sem_w):
        sgs = [sg0, sg1, sg2, sg3, sg4, sg5]
        cid = lax.axis_index("c")
        sid = lax.axis_index("s")
        wid = sid * _N_CORES + cid
        base = jnp.minimum(wid * chunk, n_students - chunk)
        base = pl.multiple_of(base, _LANES)

        lane = lax.iota(jnp.int32, _LANES)
        ninf = jnp.float32(-jnp.inf)

        pltpu.async_copy(sf_hbm, sf_v, sem_s).wait()
        fv = sf_v[...]
        lam = jnp.full((_LANES,), jnp.max(jnp.where(lane == 0, fv, ninf)))
        gp = jnp.full((_LANES,), jnp.max(jnp.where(lane == 1, fv, ninf)))
        r = jnp.max(jnp.where(lane == 2, fv, ninf)).astype(jnp.int32)
        c = jnp.max(jnp.where(lane == 3, fv, ninf)).astype(jnp.int32)

        cp_d = pltpu.make_async_copy(D_hbm.at[r], drow_v, sem_d)
        cp_d.start()

        idx0 = (base + lane) * n_concepts + c
        copies = []
        row = 0
        for b, nr in enumerate(batches):
            n_el = nr * _IDX_W
            off = row * _IDX_W

            @plsc.parallel_loop(off // _LANES, (off + n_el) // _LANES,
                                unroll=8)
            def _build(g):
                j0 = g * _LANES
                idx_v[pl.ds(j0, _LANES)] = idx0 + j0 * n_concepts

            cp = pltpu.make_async_copy(
                A_hbm.at[idx_v.at[pl.ds(off, n_el)]],
                a_v.at[pl.ds(off, n_el)],
                sgs[b])
            cp.start()
            copies.append((b, cp))
            row += nr

        cp_d.wait()
        d = plsc.load_gather(drow_v, [jnp.full((_LANES,), c, jnp.int32)])
        one = jnp.full((_LANES,), jnp.float32(1.0))
        b_vec = lam * d

        row = 0
        for b, nr in enumerate(batches):
            for bb, cp in copies:
                if bb == b:
                    cp.wait()
            off = row * _IDX_W
            n_el = nr * _IDX_W

            @plsc.parallel_loop(off // _LANES, (off + n_el) // _LANES,
                                unroll=8)
            def _compute(g):
                j0 = g * _LANES
                a = a_v[pl.ds(j0, _LANES)]
                y = jnp.maximum(gp, one - jnp.exp(b_vec - lam * a))
                y_v[pl.ds(j0, _LANES)] = y
            pltpu.make_async_copy(
                y_v.at[pl.ds(off, nr * _IDX_W)],
                out_hbm.at[pl.ds(base + off, nr * _IDX_W)],
                sem_w,
            ).start()
            row += nr

        row = 0
        for nr in batches:
            pltpu.make_async_copy(
                y_v.at[pl.ds(row * _IDX_W, nr * _IDX_W)],
                out_hbm.at[pl.ds(base + row * _IDX_W, nr * _IDX_W)],
                sem_w,
            ).wait()
            row += nr

    return run(A_flat, D, scal_f)
